# baseline (speedup 1.0000x reference)
import functools

import jax
import jax.numpy as jnp
import numpy as np
from jax import lax
from jax.experimental import pallas as pl
from jax.experimental.pallas import tpu as pltpu
from jax.experimental.pallas import tpu_sc as plsc

RADIUS = 0.02
MAX_NB = 10
N_TAPS = 3
N_LAYERS = 2

_NCELL = 49
_SCALE = np.float32(_NCELL)
_R2 = np.float32(RADIUS * RADIUS)
_QP = 10240
_RPT = 320
_NC = 2
_NW = 32
_HCAP = 512


def _sc_mesh():
    return plsc.VectorSubcoreMesh(core_axis_name="c", subcore_axis_name="s")


@functools.lru_cache(maxsize=None)
def _make_search(PP, CSP):
    @functools.partial(
        pl.kernel,
        out_type=jax.ShapeDtypeStruct((_QP * 16,), jnp.int32),
        mesh=_sc_mesh(),
        compiler_params=pltpu.CompilerParams(needs_layout_passes=False),
        scratch_types=[
            pltpu.VMEM((_RPT + 16,), jnp.float32),
            pltpu.VMEM((_RPT + 16,), jnp.float32),
            pltpu.VMEM((_RPT + 16,), jnp.int32),
            pltpu.VMEM((_RPT + 16,), jnp.int32),
            pltpu.VMEM((PP,), jnp.float32),
            pltpu.VMEM((PP,), jnp.float32),
            pltpu.VMEM((PP,), jnp.int32),
            pltpu.VMEM((CSP,), jnp.int32),
            pltpu.VMEM((_HCAP + 32,), jnp.float32),
            pltpu.VMEM((_HCAP + 32,), jnp.int32),
            pltpu.VMEM((_RPT * 16,), jnp.int32),
        ],
    )
    def search(qx_h, qy_h, sx_h, sy_h, pid_h, cs_h, out_h,
               qx_v, qy_v, cx_v, cy_v, sx_v, sy_v, pid_v, cs_v,
               hd2, hpid, out_v):
        wid = lax.axis_index("s") * _NC + lax.axis_index("c")
        base = wid * _RPT
        pltpu.sync_copy(qx_h.at[pl.ds(base, _RPT)], qx_v.at[pl.ds(0, _RPT)])
        pltpu.sync_copy(qy_h.at[pl.ds(base, _RPT)], qy_v.at[pl.ds(0, _RPT)])
        pltpu.sync_copy(sx_h, sx_v)
        pltpu.sync_copy(sy_h, sy_v)
        pltpu.sync_copy(pid_h, pid_v)
        pltpu.sync_copy(cs_h, cs_v)

        ic = lax.iota(jnp.int32, 16)

        def cell_body(i, carry):
            o = i * 16
            xs = qx_v[pl.ds(o, 16)]
            ys = qy_v[pl.ds(o, 16)]
            cx_v[pl.ds(o, 16)] = jnp.clip((xs * _SCALE).astype(jnp.int32),
                                          0, _NCELL - 1)
            cy_v[pl.ds(o, 16)] = jnp.clip((ys * _SCALE).astype(jnp.int32),
                                          0, _NCELL - 1)
            return carry
        lax.fori_loop(0, _RPT // 16, cell_body, jnp.int32(0))

        def row_body(r, carry):
            gr = base + r
            cx = cx_v[pl.ds(r, 16)][0]
            cy = cy_v[pl.ds(r, 16)][0]
            qxs = jnp.broadcast_to(qx_v[pl.ds(r, 16)][0], (16,))
            qys = jnp.broadcast_to(qy_v[pl.ds(r, 16)][0], (16,))
            ylo = jnp.maximum(cy - 1, 0)
            yhi = jnp.minimum(cy + 1, _NCELL - 1)

            hcnt = jnp.int32(0)
            for dxc in (-1, 0, 1):
                ax = cx + dxc
                ok = jnp.logical_and(ax >= 0, ax < _NCELL)
                c0 = jnp.where(ok, ax * _NCELL + ylo, 0)
                c1 = jnp.where(ok, ax * _NCELL + yhi, 0)
                sbeg = cs_v[pl.ds(c0, 16)][0]
                send = jnp.where(ok, cs_v[pl.ds(c1 + 1, 16)][0], sbeg)
                ntr = (send - sbeg + 15) // 16

                def chunk_body(j, hc, sbeg=sbeg, send=send, qxs=qxs,
                               qys=qys, gr=gr):
                    o = sbeg + j * 16
                    xs = sx_v[pl.ds(o, 16)]
                    ys = sy_v[pl.ds(o, 16)]
                    pv = pid_v[pl.ds(o, 16)]
                    lane = ic + o
                    dx = xs - qxs
                    dy = ys - qys
                    d2 = dx * dx + dy * dy
                    m = (lane < send) & (d2 <= _R2) & (pv != gr)
                    hc = jnp.minimum(hc, _HCAP)
                    csum = plsc.cumsum(jnp.where(m, 1, 0))
                    pos = jnp.where(m, hc - 1 + csum, _HCAP + 16)
                    plsc.store_scatter(hd2, [pos], d2)
                    plsc.store_scatter(hpid, [pos], pv)
                    return hc + csum[15]
                hcnt = lax.fori_loop(0, ntr, chunk_body, hcnt)

            hcnt = jnp.minimum(hcnt, _HCAP)
            nv = (hcnt + 15) // 16
            out_vec = jnp.full((16,), -1, jnp.int32)
            for s_ in range(MAX_NB):
                def mn_body(j, mv):
                    o = j * 16
                    dv = hd2[pl.ds(o, 16)]
                    return jnp.minimum(mv, jnp.where(ic + o < hcnt, dv,
                                                     jnp.inf))
                mvec = lax.fori_loop(0, nv, mn_body,
                                     jnp.full((16,), jnp.inf, jnp.float32))
                mb = jnp.broadcast_to(jnp.min(mvec), (16,))

                def sel_body(j, sv):
                    o = j * 16
                    dv = hd2[pl.ds(o, 16)]
                    dv = jnp.where(ic + o < hcnt, dv, jnp.inf)
                    pv = hpid[pl.ds(o, 16)]
                    return jnp.minimum(sv, jnp.where(dv == mb, pv,
                                                     jnp.int32(2 ** 30)))
                svec = lax.fori_loop(0, nv, sel_body,
                                     jnp.full((16,), 2 ** 30, jnp.int32))
                sb = jnp.broadcast_to(jnp.min(svec), (16,))

                def clr_body(j, carry2):
                    o = j * 16
                    dv = hd2[pl.ds(o, 16)]
                    pv = hpid[pl.ds(o, 16)]
                    hd2[pl.ds(o, 16)] = jnp.where(pv == sb, jnp.inf, dv)
                    return carry2
                lax.fori_loop(0, nv, clr_body, jnp.int32(0))
                out_vec = jnp.where((ic == s_) & (mb < jnp.inf), sb, out_vec)
            out_v[pl.ds(r * 16, 16)] = out_vec
            return carry
        lax.fori_loop(0, _RPT, row_body, jnp.int32(0))
        pltpu.sync_copy(out_v, out_h.at[pl.ds(base * 16, _RPT * 16)])

    return search


def _bin_points(px, py):
    P = px.shape[0]
    cx = jnp.clip((px * _SCALE).astype(jnp.int32), 0, _NCELL - 1)
    cy = jnp.clip((py * _SCALE).astype(jnp.int32), 0, _NCELL - 1)
    cid = cx * _NCELL + cy
    order = jnp.argsort(cid)
    cid_s = cid[order]
    nc2 = _NCELL * _NCELL
    cs = jnp.searchsorted(cid_s, jnp.arange(nc2 + 1, dtype=jnp.int32),
                          side='left').astype(jnp.int32)
    PP = ((P + 16 + 15) // 16) * 16
    pad = PP - P
    sx = jnp.concatenate([px[order], jnp.full((pad,), 1e9, jnp.float32)])
    sy = jnp.concatenate([py[order], jnp.full((pad,), 1e9, jnp.float32)])
    pid = jnp.concatenate([order.astype(jnp.int32),
                           jnp.full((pad,), -2, jnp.int32)])
    CSP = ((nc2 + 1 + 16 + 15) // 16) * 16
    cs = jnp.concatenate([cs, jnp.full((CSP - nc2 - 1,), P, jnp.int32)])
    return sx, sy, pid, cs, PP, CSP


def _pad_queries(q):
    n = q.shape[0]
    return jnp.concatenate([q, jnp.full((_QP - n,), 0.5, jnp.float32)])


def _radius_search(qx, qy, tpx, tpy):
    sx, sy, pid, cs, PP, CSP = _bin_points(tpx, tpy)
    out = _make_search(PP, CSP)(qx, qy, sx, sy, pid, cs)
    return out.reshape(_QP, 16)


@functools.lru_cache(maxsize=None)
def _make_gat(P, vmin):
    NEG = np.float32(np.finfo(np.float32).min)

    @functools.partial(
        pl.kernel,
        out_type=jax.ShapeDtypeStruct((_QP * 16,), jnp.float32),
        mesh=_sc_mesh(),
        compiler_params=pltpu.CompilerParams(needs_layout_passes=False),
        scratch_types=[
            pltpu.VMEM((_RPT * 16,), jnp.int32),
            pltpu.VMEM((_RPT + 16,), jnp.float32),
            pltpu.VMEM((_RPT + 16,), jnp.float32),
            pltpu.VMEM((P,), jnp.float32),
            pltpu.VMEM((P,), jnp.float32),
            pltpu.VMEM((96,), jnp.float32),
            pltpu.VMEM((_RPT * 16,), jnp.float32),
        ],
    )
    def gat(nbr_h, qx_h, qy_h, px_h, py_h, w_h, out_h,
            nbr_v, qx_v, qy_v, px_v, py_v, w_v, out_v):
        wid = lax.axis_index("s") * _NC + lax.axis_index("c")
        base = wid * _RPT
        pltpu.sync_copy(nbr_h.at[pl.ds(base * 16, _RPT * 16)], nbr_v)
        pltpu.sync_copy(qx_h.at[pl.ds(base, _RPT)], qx_v.at[pl.ds(0, _RPT)])
        pltpu.sync_copy(qy_h.at[pl.ds(base, _RPT)], qy_v.at[pl.ds(0, _RPT)])
        pltpu.sync_copy(px_h, px_v)
        pltpu.sync_copy(py_h, py_v)
        pltpu.sync_copy(w_h, w_v)
        ic = lax.iota(jnp.int32, 16)

        def bfrn(v):
            u = plsc.bitcast(v, jnp.int32)
            r = (u + 32767 + ((u >> 16) & 1)) & jnp.int32(-65536)
            return plsc.bitcast(r, jnp.float32)

        ws0 = bfrn(w_v[pl.ds(0, 16)])
        ws1 = bfrn(w_v[pl.ds(16, 16)])
        wt0 = bfrn(w_v[pl.ds(32, 16)])
        wt1 = bfrn(w_v[pl.ds(48, 16)])
        att = bfrn(w_v[pl.ds(64, 16)])
        bias = w_v[pl.ds(80, 16)]

        def row_body(r, carry):
            nb = nbr_v[pl.ds(r * 16, 16)]
            validm = nb >= vmin
            nbc = jnp.where(validm, nb, 0)
            gx = bfrn(plsc.load_gather(px_v, [nbc]))
            gy = bfrn(plsc.load_gather(py_v, [nbc]))
            qxs = bfrn(jnp.broadcast_to(qx_v[pl.ds(r, 16)][0], (16,)))
            qys = bfrn(jnp.broadcast_to(qy_v[pl.ds(r, 16)][0], (16,)))
            xt = qxs * wt0 + qys * wt1
            evec = jnp.full((16,), NEG, jnp.float32)
            for k in range(MAX_NB):
                gxk = jnp.broadcast_to(jnp.sum(jnp.where(ic == k, gx, 0.0)), (16,))
                gyk = jnp.broadcast_to(jnp.sum(jnp.where(ic == k, gy, 0.0)), (16,))
                gk = gxk * ws0 + gyk * ws1
                u = xt + gk
                lr = bfrn(jnp.where(u >= 0, u, 0.2 * u))
                ek = jnp.sum(lr * att)
                evec = jnp.where(ic == k, ek, evec)
            evec = jnp.where(validm, evec, NEG)
            emax = jnp.max(evec)
            ex = jnp.where(validm, jnp.exp(evec - emax), 0.0)
            den = jnp.broadcast_to(jnp.sum(ex), (16,))
            alpha = ex / jnp.maximum(den, 1e-12)
            acc = jnp.zeros((16,), jnp.float32)
            for k in range(MAX_NB):
                ak = jnp.broadcast_to(jnp.sum(jnp.where(ic == k, alpha, 0.0)), (16,))
                gxk = jnp.broadcast_to(jnp.sum(jnp.where(ic == k, gx, 0.0)), (16,))
                gyk = jnp.broadcast_to(jnp.sum(jnp.where(ic == k, gy, 0.0)), (16,))
                acc = acc + ak * (gxk * ws0 + gyk * ws1)
            out_v[pl.ds(r * 16, 16)] = acc + bias
            return carry
        lax.fori_loop(0, _RPT, row_body, jnp.int32(0))
        pltpu.sync_copy(out_v, out_h.at[pl.ds(base * 16, _RPT * 16)])

    return gat


def _gat_agg(nbr_full, qx, qy, tpx, tpy, Ws, Wt, att, bias, vmin):
    P = tpx.shape[0]
    w6 = jnp.concatenate([Ws[0], Ws[1], Wt[0], Wt[1], att, bias])
    out = _make_gat(P, vmin)(nbr_full.reshape(-1), qx, qy, tpx, tpy, w6)
    return out.reshape(_QP, 16)


_ECH = 128
_NCHUNK = 40
_ACC_ROWS = 10368


@functools.lru_cache(maxsize=None)
def _make_shift(n_in):
    outs = (jax.ShapeDtypeStruct((10240, 64), jnp.float32),
            jax.ShapeDtypeStruct((10240, 64), jnp.float32))
    scratch = [
        pltpu.VMEM((_NCHUNK, _ECH), jnp.int32),
        pltpu.VMEM((_NCHUNK, _ECH), jnp.int32),
    ]
    for _ in range(4 * n_in):
        scratch += [pltpu.VMEM((_ECH, 64), jnp.float32)]
    scratch += [
        pltpu.VMEM_SHARED((_ACC_ROWS, 64), jnp.float32),
        pltpu.SemaphoreType.DMA,
        pltpu.SemaphoreType.DMA,
    ]

    @functools.partial(
        pl.kernel,
        out_type=outs,
        mesh=_sc_mesh(),
        compiler_params=pltpu.CompilerParams(needs_layout_passes=False,
                                             use_tc_tiling_on_sc=False),
        scratch_types=scratch,
    )
    def shift(*refs):
        ys = refs[:n_in]
        src_h, dst_h, zeros_h = refs[n_in:n_in + 3]
        yA, yB = refs[n_in + 3:n_in + 5]
        src_v, dst_v = refs[n_in + 5:n_in + 7]
        bufs = refs[n_in + 7:n_in + 7 + 4 * n_in]
        acc = refs[n_in + 7 + 4 * n_in]
        sem0, sem1 = refs[-2], refs[-1]
        cidx = lax.axis_index("c")
        sidx = lax.axis_index("s")
        wid = sidx * _NC + cidx
        pltpu.sync_copy(src_h.at[wid], src_v)
        pltpu.sync_copy(dst_h.at[wid], dst_v)
        pltpu.sync_copy(zeros_h, acc.at[pl.ds(sidx * 640, 640)])
        plsc.subcore_barrier()
        NB = 4
        for yi in range(n_in):
            y_h = ys[yi]
            bs = bufs[NB * yi:NB * yi + NB]

            def group(g, carry, y_h=y_h, bs=bs):
                for i in range(NB):
                    pltpu.async_copy(y_h.at[src_v.at[g * NB + i]], bs[i], sem0)
                for i in range(NB):
                    pltpu.make_async_copy(y_h.at[src_v.at[0]], bs[i],
                                          sem0).wait()
                for i in range(NB):
                    pltpu.async_copy(bs[i], acc.at[dst_v.at[g * NB + i]],
                                     sem1, add=True)
                for i in range(NB):
                    pltpu.make_async_copy(bs[i], acc.at[dst_v.at[0]],
                                          sem1).wait()
                return carry
            lax.fori_loop(0, _NCHUNK // NB, group, jnp.int32(0))
        plsc.subcore_barrier()

        @pl.when(cidx == 0)
        def _():
            pltpu.sync_copy(acc.at[pl.ds(sidx * 640, 640)],
                            yA.at[pl.ds(sidx * 640, 640)])

        @pl.when(cidx == 1)
        def _():
            pltpu.sync_copy(acc.at[pl.ds(sidx * 640, 640)],
                            yB.at[pl.ds(sidx * 640, 640)])

    return shift


def _prep_edges(src, dst):
    E = src.shape[0]
    tot = _NW * _NCHUNK * _ECH
    pads = tot - E
    src_p = jnp.concatenate([src.astype(jnp.int32),
                             jnp.zeros((pads,), jnp.int32)])
    dst_p = jnp.concatenate([dst.astype(jnp.int32),
                             jnp.full((pads,), 10240, jnp.int32)])
    return (src_p.reshape(_NW, _NCHUNK, _ECH),
            dst_p.reshape(_NW, _NCHUNK, _ECH))


def _shift(ys, src3, dst3, zeros640):
    return _make_shift(len(ys))(*ys, src3, dst3, zeros640)


def _bx_body(own_ref, ws_ref, bs_ref, a_ref, t_ref, o_ref):
    s = jnp.dot(own_ref[...], ws_ref[...],
                preferred_element_type=jnp.float32) + bs_ref[...]
    o_ref[...] = jnp.concatenate([s, a_ref[...], t_ref[...]], axis=1)


def _build_x(own_obs, W_self, b_self, a, t):
    N = own_obs.shape[0]
    return pl.pallas_call(
        _bx_body,
        out_shape=jax.ShapeDtypeStruct((N, 64), jnp.float32),
    )(own_obs, W_self, b_self.reshape(1, -1), a, t)


def _layer_body(x_ref, a1, b1, a2, b2, a3, b3, w_ref, bf_ref, g_ref, be_ref,
                o_ref):
    x = x_ref[...]
    W = w_ref[...]
    h = jnp.dot(x, W[:64], preferred_element_type=jnp.float32) + bf_ref[...]
    for i, (pa, pb) in enumerate(((a1, b1), (a2, b2), (a3, b3))):
        y = pa[...][:10000] + pb[...][:10000]
        h = h + jnp.dot(y, W[64 * (i + 1):64 * (i + 2)],
                        preferred_element_type=jnp.float32)
    mean = jnp.mean(h, axis=0, keepdims=True)
    var = jnp.mean((h - mean) ** 2, axis=0, keepdims=True)
    hn = (h - mean) / jnp.sqrt(var + 1e-5) * g_ref[...] + be_ref[...]
    o_ref[...] = x + jnp.where(hn >= 0, hn, 0.01 * hn)


def _layer(x, parts, Wl, bfl, gl, bel):
    N = x.shape[0]
    return pl.pallas_call(
        _layer_body,
        out_shape=jax.ShapeDtypeStruct((N, 64), jnp.float32),
    )(x, *parts, Wl, bfl.reshape(1, -1), gl.reshape(1, -1), bel.reshape(1, -1))


def _out_mm_body(x_ref, w_ref, b_ref, o_ref):
    o_ref[...] = jnp.dot(x_ref[...], w_ref[...],
                         preferred_element_type=jnp.float32) + b_ref[...]


def _out_matmul(x, W, b):
    N = x.shape[0]
    OUT = W.shape[1]
    Wp = jnp.zeros((W.shape[0], 128), jnp.float32).at[:, :OUT].set(W)
    bp = jnp.zeros((1, 128), jnp.float32).at[0, :OUT].set(b)
    out = pl.pallas_call(
        _out_mm_body,
        out_shape=jax.ShapeDtypeStruct((N, 128), jnp.float32),
    )(x, Wp, bp)
    return out[:, :OUT]


def kernel(own_obs, agent_positions, target_positions, edge_index, Ws_a, Wt_a, att_a, b_a, Ws_t, Wt_t, att_t, b_t, W_self, b_self,
           Wf, bf, gamma, beta, W_out, b_out):
    N = own_obs.shape[0]
    qx = _pad_queries(agent_positions[:, 0])
    qy = _pad_queries(agent_positions[:, 1])
    pos_all = jnp.concatenate([agent_positions, target_positions], 0)

    nbrA = _radius_search(qx, qy, agent_positions[:, 0], agent_positions[:, 1])
    nbrC = _radius_search(qx, qy, pos_all[:, 0], pos_all[:, 1])

    a = _gat_agg(nbrA, qx, qy, agent_positions[:, 0], agent_positions[:, 1],
                 Ws_a, Wt_a, att_a, b_a, 0)[:N]
    t = _gat_agg(nbrC, qx, qy, pos_all[:, 0], pos_all[:, 1],
                 Ws_t, Wt_t, att_t, b_t, N)[:N]
    x = _build_x(own_obs, W_self, b_self, a, t)

    src3, dst3 = _prep_edges(edge_index[0], edge_index[1])
    zeros640 = jnp.zeros((640, 64), jnp.float32)
    Wcat = Wf.reshape(N_LAYERS, (N_TAPS + 1) * 64, 64)
    for l in range(N_LAYERS):
        yA1, yB1 = _shift([x], src3, dst3, zeros640)
        yA2, yB2 = _shift([yA1, yB1], src3, dst3, zeros640)
        yA3, yB3 = _shift([yA2, yB2], src3, dst3, zeros640)
        x = _layer(x, (yA1, yB1, yA2, yB2, yA3, yB3),
                   Wcat[l], bf[l], gamma[l], beta[l])
    return _out_matmul(x, W_out, b_out)


# speedup vs baseline: 1.0536x; 1.0536x over previous
import functools

import jax
import jax.numpy as jnp
import numpy as np
from jax import lax
from jax.experimental import pallas as pl
from jax.experimental.pallas import tpu as pltpu
from jax.experimental.pallas import tpu_sc as plsc

RADIUS = 0.02
MAX_NB = 10
N_TAPS = 3
N_LAYERS = 2

_NCELL = 49
_SCALE = np.float32(_NCELL)
_R2 = np.float32(RADIUS * RADIUS)
_QP = 10240
_RPT = 320
_NC = 2
_NW = 32
_HCAP = 512


def _sc_mesh():
    return plsc.VectorSubcoreMesh(core_axis_name="c", subcore_axis_name="s")


@functools.lru_cache(maxsize=None)
def _make_search(PP, CSP):
    @functools.partial(
        pl.kernel,
        out_type=jax.ShapeDtypeStruct((_QP * 16,), jnp.int32),
        mesh=_sc_mesh(),
        compiler_params=pltpu.CompilerParams(needs_layout_passes=False),
        scratch_types=[
            pltpu.VMEM((_RPT + 16,), jnp.float32),
            pltpu.VMEM((_RPT + 16,), jnp.float32),
            pltpu.VMEM((_RPT + 16,), jnp.int32),
            pltpu.VMEM((_RPT + 16,), jnp.int32),
            pltpu.VMEM((PP,), jnp.float32),
            pltpu.VMEM((PP,), jnp.float32),
            pltpu.VMEM((PP,), jnp.int32),
            pltpu.VMEM((CSP,), jnp.int32),
            pltpu.VMEM((_HCAP + 32,), jnp.float32),
            pltpu.VMEM((_HCAP + 32,), jnp.int32),
            pltpu.VMEM((_RPT * 16,), jnp.int32),
        ],
    )
    def search(qx_h, qy_h, sx_h, sy_h, pid_h, cs_h, out_h,
               qx_v, qy_v, cx_v, cy_v, sx_v, sy_v, pid_v, cs_v,
               hd2, hpid, out_v):
        wid = lax.axis_index("s") * _NC + lax.axis_index("c")
        base = wid * _RPT
        pltpu.sync_copy(qx_h.at[pl.ds(base, _RPT)], qx_v.at[pl.ds(0, _RPT)])
        pltpu.sync_copy(qy_h.at[pl.ds(base, _RPT)], qy_v.at[pl.ds(0, _RPT)])
        pltpu.sync_copy(sx_h, sx_v)
        pltpu.sync_copy(sy_h, sy_v)
        pltpu.sync_copy(pid_h, pid_v)
        pltpu.sync_copy(cs_h, cs_v)

        ic = lax.iota(jnp.int32, 16)

        def cell_body(i, carry):
            o = i * 16
            xs = qx_v[pl.ds(o, 16)]
            ys = qy_v[pl.ds(o, 16)]
            cx_v[pl.ds(o, 16)] = jnp.clip((xs * _SCALE).astype(jnp.int32),
                                          0, _NCELL - 1)
            cy_v[pl.ds(o, 16)] = jnp.clip((ys * _SCALE).astype(jnp.int32),
                                          0, _NCELL - 1)
            return carry
        lax.fori_loop(0, _RPT // 16, cell_body, jnp.int32(0))

        def row_body(r, carry):
            gr = base + r
            cx = cx_v[pl.ds(r, 16)][0]
            cy = cy_v[pl.ds(r, 16)][0]
            qxs = jnp.broadcast_to(qx_v[pl.ds(r, 16)][0], (16,))
            qys = jnp.broadcast_to(qy_v[pl.ds(r, 16)][0], (16,))
            ylo = jnp.maximum(cy - 1, 0)
            yhi = jnp.minimum(cy + 1, _NCELL - 1)

            hcnt = jnp.int32(0)
            for dxc in (-1, 0, 1):
                ax = cx + dxc
                ok = jnp.logical_and(ax >= 0, ax < _NCELL)
                c0 = jnp.where(ok, ax * _NCELL + ylo, 0)
                c1 = jnp.where(ok, ax * _NCELL + yhi, 0)
                sbeg = cs_v[pl.ds(c0, 16)][0]
                send = jnp.where(ok, cs_v[pl.ds(c1 + 1, 16)][0], sbeg)
                ntr = (send - sbeg + 15) // 16

                def chunk_body(j, hc, sbeg=sbeg, send=send, qxs=qxs,
                               qys=qys, gr=gr):
                    o = sbeg + j * 16
                    xs = sx_v[pl.ds(o, 16)]
                    ys = sy_v[pl.ds(o, 16)]
                    pv = pid_v[pl.ds(o, 16)]
                    lane = ic + o
                    dx = xs - qxs
                    dy = ys - qys
                    d2 = dx * dx + dy * dy
                    m = (lane < send) & (d2 <= _R2) & (pv != gr)
                    hc = jnp.minimum(hc, _HCAP)
                    csum = plsc.cumsum(jnp.where(m, 1, 0))
                    pos = jnp.where(m, hc - 1 + csum, _HCAP + 16)
                    plsc.store_scatter(hd2, [pos], d2)
                    plsc.store_scatter(hpid, [pos], pv)
                    return hc + csum[15]
                hcnt = lax.fori_loop(0, ntr, chunk_body, hcnt)

            hcnt = jnp.minimum(hcnt, _HCAP)
            nv = (hcnt + 15) // 16
            out_vec = jnp.full((16,), -1, jnp.int32)
            for s_ in range(MAX_NB):
                def mn_body(j, mv):
                    o = j * 16
                    dv = hd2[pl.ds(o, 16)]
                    return jnp.minimum(mv, jnp.where(ic + o < hcnt, dv,
                                                     jnp.inf))
                mvec = lax.fori_loop(0, nv, mn_body,
                                     jnp.full((16,), jnp.inf, jnp.float32))
                mb = jnp.broadcast_to(jnp.min(mvec), (16,))

                def sel_body(j, sv):
                    o = j * 16
                    dv = hd2[pl.ds(o, 16)]
                    dv = jnp.where(ic + o < hcnt, dv, jnp.inf)
                    pv = hpid[pl.ds(o, 16)]
                    return jnp.minimum(sv, jnp.where(dv == mb, pv,
                                                     jnp.int32(2 ** 30)))
                svec = lax.fori_loop(0, nv, sel_body,
                                     jnp.full((16,), 2 ** 30, jnp.int32))
                sb = jnp.broadcast_to(jnp.min(svec), (16,))

                def clr_body(j, carry2):
                    o = j * 16
                    dv = hd2[pl.ds(o, 16)]
                    pv = hpid[pl.ds(o, 16)]
                    hd2[pl.ds(o, 16)] = jnp.where(pv == sb, jnp.inf, dv)
                    return carry2
                lax.fori_loop(0, nv, clr_body, jnp.int32(0))
                out_vec = jnp.where((ic == s_) & (mb < jnp.inf), sb, out_vec)
            out_v[pl.ds(r * 16, 16)] = out_vec
            return carry
        lax.fori_loop(0, _RPT, row_body, jnp.int32(0))
        pltpu.sync_copy(out_v, out_h.at[pl.ds(base * 16, _RPT * 16)])

    return search


def _bin_points(px, py):
    P = px.shape[0]
    cx = jnp.clip((px * _SCALE).astype(jnp.int32), 0, _NCELL - 1)
    cy = jnp.clip((py * _SCALE).astype(jnp.int32), 0, _NCELL - 1)
    cid = cx * _NCELL + cy
    order = jnp.argsort(cid)
    cid_s = cid[order]
    nc2 = _NCELL * _NCELL
    cs = jnp.searchsorted(cid_s, jnp.arange(nc2 + 1, dtype=jnp.int32),
                          side='left').astype(jnp.int32)
    PP = ((P + 16 + 15) // 16) * 16
    pad = PP - P
    sx = jnp.concatenate([px[order], jnp.full((pad,), 1e9, jnp.float32)])
    sy = jnp.concatenate([py[order], jnp.full((pad,), 1e9, jnp.float32)])
    pid = jnp.concatenate([order.astype(jnp.int32),
                           jnp.full((pad,), -2, jnp.int32)])
    CSP = ((nc2 + 1 + 16 + 15) // 16) * 16
    cs = jnp.concatenate([cs, jnp.full((CSP - nc2 - 1,), P, jnp.int32)])
    return sx, sy, pid, cs, PP, CSP


def _pad_queries(q):
    n = q.shape[0]
    return jnp.concatenate([q, jnp.full((_QP - n,), 0.5, jnp.float32)])


def _radius_search(qx, qy, tpx, tpy):
    sx, sy, pid, cs, PP, CSP = _bin_points(tpx, tpy)
    out = _make_search(PP, CSP)(qx, qy, sx, sy, pid, cs)
    return out.reshape(_QP, 16)


@functools.lru_cache(maxsize=None)
def _make_gat(P, vmin):
    NEG = np.float32(np.finfo(np.float32).min)

    @functools.partial(
        pl.kernel,
        out_type=jax.ShapeDtypeStruct((_QP * 16,), jnp.float32),
        mesh=_sc_mesh(),
        compiler_params=pltpu.CompilerParams(needs_layout_passes=False),
        scratch_types=[
            pltpu.VMEM((_RPT * 16,), jnp.int32),
            pltpu.VMEM((_RPT + 16,), jnp.float32),
            pltpu.VMEM((_RPT + 16,), jnp.float32),
            pltpu.VMEM((P,), jnp.float32),
            pltpu.VMEM((P,), jnp.float32),
            pltpu.VMEM((96,), jnp.float32),
            pltpu.VMEM((_RPT * 16,), jnp.float32),
        ],
    )
    def gat(nbr_h, qx_h, qy_h, px_h, py_h, w_h, out_h,
            nbr_v, qx_v, qy_v, px_v, py_v, w_v, out_v):
        wid = lax.axis_index("s") * _NC + lax.axis_index("c")
        base = wid * _RPT
        pltpu.sync_copy(nbr_h.at[pl.ds(base * 16, _RPT * 16)], nbr_v)
        pltpu.sync_copy(qx_h.at[pl.ds(base, _RPT)], qx_v.at[pl.ds(0, _RPT)])
        pltpu.sync_copy(qy_h.at[pl.ds(base, _RPT)], qy_v.at[pl.ds(0, _RPT)])
        pltpu.sync_copy(px_h, px_v)
        pltpu.sync_copy(py_h, py_v)
        pltpu.sync_copy(w_h, w_v)
        ic = lax.iota(jnp.int32, 16)

        def bfrn(v):
            u = plsc.bitcast(v, jnp.int32)
            r = (u + 32767 + ((u >> 16) & 1)) & jnp.int32(-65536)
            return plsc.bitcast(r, jnp.float32)

        ws0 = bfrn(w_v[pl.ds(0, 16)])
        ws1 = bfrn(w_v[pl.ds(16, 16)])
        wt0 = bfrn(w_v[pl.ds(32, 16)])
        wt1 = bfrn(w_v[pl.ds(48, 16)])
        att = bfrn(w_v[pl.ds(64, 16)])
        bias = w_v[pl.ds(80, 16)]

        def row_body(r, carry):
            nb = nbr_v[pl.ds(r * 16, 16)]
            validm = nb >= vmin
            nbc = jnp.where(validm, nb, 0)
            gx = bfrn(plsc.load_gather(px_v, [nbc]))
            gy = bfrn(plsc.load_gather(py_v, [nbc]))
            qxs = bfrn(jnp.broadcast_to(qx_v[pl.ds(r, 16)][0], (16,)))
            qys = bfrn(jnp.broadcast_to(qy_v[pl.ds(r, 16)][0], (16,)))
            xt = qxs * wt0 + qys * wt1
            evec = jnp.full((16,), NEG, jnp.float32)
            for k in range(MAX_NB):
                gxk = jnp.broadcast_to(jnp.sum(jnp.where(ic == k, gx, 0.0)), (16,))
                gyk = jnp.broadcast_to(jnp.sum(jnp.where(ic == k, gy, 0.0)), (16,))
                gk = gxk * ws0 + gyk * ws1
                u = xt + gk
                lr = bfrn(jnp.where(u >= 0, u, 0.2 * u))
                ek = jnp.sum(lr * att)
                evec = jnp.where(ic == k, ek, evec)
            evec = jnp.where(validm, evec, NEG)
            emax = jnp.max(evec)
            ex = jnp.where(validm, jnp.exp(evec - emax), 0.0)
            den = jnp.broadcast_to(jnp.sum(ex), (16,))
            alpha = ex / jnp.maximum(den, 1e-12)
            acc = jnp.zeros((16,), jnp.float32)
            for k in range(MAX_NB):
                ak = jnp.broadcast_to(jnp.sum(jnp.where(ic == k, alpha, 0.0)), (16,))
                gxk = jnp.broadcast_to(jnp.sum(jnp.where(ic == k, gx, 0.0)), (16,))
                gyk = jnp.broadcast_to(jnp.sum(jnp.where(ic == k, gy, 0.0)), (16,))
                acc = acc + ak * (gxk * ws0 + gyk * ws1)
            out_v[pl.ds(r * 16, 16)] = acc + bias
            return carry
        lax.fori_loop(0, _RPT, row_body, jnp.int32(0))
        pltpu.sync_copy(out_v, out_h.at[pl.ds(base * 16, _RPT * 16)])

    return gat


def _gat_agg(nbr_full, qx, qy, tpx, tpy, Ws, Wt, att, bias, vmin):
    P = tpx.shape[0]
    w6 = jnp.concatenate([Ws[0], Ws[1], Wt[0], Wt[1], att, bias])
    out = _make_gat(P, vmin)(nbr_full.reshape(-1), qx, qy, tpx, tpy, w6)
    return out.reshape(_QP, 16)


_ECH = 128
_NCHUNK = 40
_ACC_ROWS = 10368


@functools.lru_cache(maxsize=None)
def _make_shift(n_in):
    outs = (jax.ShapeDtypeStruct((10240, 64), jnp.float32),
            jax.ShapeDtypeStruct((10240, 64), jnp.float32))
    scratch = [
        pltpu.VMEM((_NCHUNK, _ECH), jnp.int32),
        pltpu.VMEM((_NCHUNK, _ECH), jnp.int32),
    ]
    for _ in range(n_in):
        scratch += [pltpu.VMEM((_ECH, 64), jnp.float32),
                    pltpu.VMEM((_ECH, 64), jnp.float32)]
    scratch += [
        pltpu.VMEM_SHARED((_ACC_ROWS, 64), jnp.float32),
        pltpu.SemaphoreType.DMA,
        pltpu.SemaphoreType.DMA,
    ]

    @functools.partial(
        pl.kernel,
        out_type=outs,
        mesh=_sc_mesh(),
        compiler_params=pltpu.CompilerParams(needs_layout_passes=False,
                                             use_tc_tiling_on_sc=False),
        scratch_types=scratch,
    )
    def shift(*refs):
        ys = refs[:n_in]
        src_h, dst_h, zeros_h = refs[n_in:n_in + 3]
        yA, yB = refs[n_in + 3:n_in + 5]
        src_v, dst_v = refs[n_in + 5:n_in + 7]
        bufs = refs[n_in + 7:n_in + 7 + 2 * n_in]
        acc = refs[n_in + 7 + 2 * n_in]
        sem0, sem1 = refs[-2], refs[-1]
        cidx = lax.axis_index("c")
        sidx = lax.axis_index("s")
        wid = sidx * _NC + cidx
        pltpu.sync_copy(src_h.at[wid], src_v)
        pltpu.sync_copy(dst_h.at[wid], dst_v)
        pltpu.sync_copy(zeros_h, acc.at[pl.ds(sidx * 640, 640)])
        plsc.subcore_barrier()
        for yi in range(n_in):
            y_h = ys[yi]
            b0 = bufs[2 * yi]
            b1 = bufs[2 * yi + 1]
            pltpu.async_copy(y_h.at[src_v.at[0]], b0, sem0)
            pltpu.async_copy(y_h.at[src_v.at[1]], b1, sem1)

            def step(t, carry, y_h=y_h, b0=b0, b1=b1):
                j0 = 2 * t
                pltpu.make_async_copy(y_h.at[src_v.at[0]], b0, sem0).wait()
                pltpu.sync_copy(b0, acc.at[dst_v.at[j0]], add=True)

                @pl.when(j0 + 2 < _NCHUNK)
                def _():
                    pltpu.async_copy(y_h.at[src_v.at[j0 + 2]], b0, sem0)

                pltpu.make_async_copy(y_h.at[src_v.at[0]], b1, sem1).wait()
                pltpu.sync_copy(b1, acc.at[dst_v.at[j0 + 1]], add=True)

                @pl.when(j0 + 3 < _NCHUNK)
                def _():
                    pltpu.async_copy(y_h.at[src_v.at[j0 + 3]], b1, sem1)
                return carry
            lax.fori_loop(0, _NCHUNK // 2, step, jnp.int32(0))
        plsc.subcore_barrier()

        @pl.when(cidx == 0)
        def _():
            pltpu.sync_copy(acc.at[pl.ds(sidx * 640, 640)],
                            yA.at[pl.ds(sidx * 640, 640)])

        @pl.when(cidx == 1)
        def _():
            pltpu.sync_copy(acc.at[pl.ds(sidx * 640, 640)],
                            yB.at[pl.ds(sidx * 640, 640)])

    return shift


def _prep_edges(src, dst):
    E = src.shape[0]
    tot = _NW * _NCHUNK * _ECH
    pads = tot - E
    src_p = jnp.concatenate([src.astype(jnp.int32),
                             jnp.zeros((pads,), jnp.int32)])
    dst_p = jnp.concatenate([dst.astype(jnp.int32),
                             jnp.full((pads,), 10240, jnp.int32)])
    return (src_p.reshape(_NW, _NCHUNK, _ECH),
            dst_p.reshape(_NW, _NCHUNK, _ECH))


def _shift(ys, src3, dst3, zeros640):
    return _make_shift(len(ys))(*ys, src3, dst3, zeros640)


def _bx_body(own_ref, ws_ref, bs_ref, a_ref, t_ref, o_ref):
    s = jnp.dot(own_ref[...], ws_ref[...],
                preferred_element_type=jnp.float32) + bs_ref[...]
    o_ref[...] = jnp.concatenate([s, a_ref[...], t_ref[...]], axis=1)


def _build_x(own_obs, W_self, b_self, a, t):
    N = own_obs.shape[0]
    return pl.pallas_call(
        _bx_body,
        out_shape=jax.ShapeDtypeStruct((N, 64), jnp.float32),
    )(own_obs, W_self, b_self.reshape(1, -1), a, t)


def _layer_body(x_ref, a1, b1, a2, b2, a3, b3, w_ref, bf_ref, g_ref, be_ref,
                o_ref):
    x = x_ref[...]
    W = w_ref[...]
    h = jnp.dot(x, W[:64], preferred_element_type=jnp.float32) + bf_ref[...]
    for i, (pa, pb) in enumerate(((a1, b1), (a2, b2), (a3, b3))):
        y = pa[...][:10000] + pb[...][:10000]
        h = h + jnp.dot(y, W[64 * (i + 1):64 * (i + 2)],
                        preferred_element_type=jnp.float32)
    mean = jnp.mean(h, axis=0, keepdims=True)
    var = jnp.mean((h - mean) ** 2, axis=0, keepdims=True)
    hn = (h - mean) / jnp.sqrt(var + 1e-5) * g_ref[...] + be_ref[...]
    o_ref[...] = x + jnp.where(hn >= 0, hn, 0.01 * hn)


def _layer(x, parts, Wl, bfl, gl, bel):
    N = x.shape[0]
    return pl.pallas_call(
        _layer_body,
        out_shape=jax.ShapeDtypeStruct((N, 64), jnp.float32),
    )(x, *parts, Wl, bfl.reshape(1, -1), gl.reshape(1, -1), bel.reshape(1, -1))


def _out_mm_body(x_ref, w_ref, b_ref, o_ref):
    o_ref[...] = jnp.dot(x_ref[...], w_ref[...],
                         preferred_element_type=jnp.float32) + b_ref[...]


def _out_matmul(x, W, b):
    N = x.shape[0]
    OUT = W.shape[1]
    Wp = jnp.zeros((W.shape[0], 128), jnp.float32).at[:, :OUT].set(W)
    bp = jnp.zeros((1, 128), jnp.float32).at[0, :OUT].set(b)
    out = pl.pallas_call(
        _out_mm_body,
        out_shape=jax.ShapeDtypeStruct((N, 128), jnp.float32),
    )(x, Wp, bp)
    return out[:, :OUT]


def kernel(own_obs, agent_positions, target_positions, edge_index, Ws_a, Wt_a, att_a, b_a, Ws_t, Wt_t, att_t, b_t, W_self, b_self,
           Wf, bf, gamma, beta, W_out, b_out):
    N = own_obs.shape[0]
    qx = _pad_queries(agent_positions[:, 0])
    qy = _pad_queries(agent_positions[:, 1])
    pos_all = jnp.concatenate([agent_positions, target_positions], 0)

    nbrA = _radius_search(qx, qy, agent_positions[:, 0], agent_positions[:, 1])
    nbrC = _radius_search(qx, qy, pos_all[:, 0], pos_all[:, 1])

    a = _gat_agg(nbrA, qx, qy, agent_positions[:, 0], agent_positions[:, 1],
                 Ws_a, Wt_a, att_a, b_a, 0)[:N]
    t = _gat_agg(nbrC, qx, qy, pos_all[:, 0], pos_all[:, 1],
                 Ws_t, Wt_t, att_t, b_t, N)[:N]
    x = _build_x(own_obs, W_self, b_self, a, t)

    src3, dst3 = _prep_edges(edge_index[0], edge_index[1])
    zeros640 = jnp.zeros((640, 64), jnp.float32)
    Wcat = Wf.reshape(N_LAYERS, (N_TAPS + 1) * 64, 64)
    for l in range(N_LAYERS):
        yA1, yB1 = _shift([x], src3, dst3, zeros640)
        yA2, yB2 = _shift([yA1, yB1], src3, dst3, zeros640)
        yA3, yB3 = _shift([yA2, yB2], src3, dst3, zeros640)
        x = _layer(x, (yA1, yB1, yA2, yB2, yA3, yB3),
                   Wcat[l], bf[l], gamma[l], beta[l])
    return _out_matmul(x, W_out, b_out)
